# baseline (speedup 1.0000x reference)
import jax
import jax.numpy as jnp
from jax.experimental import pallas as pl
from jax.experimental.pallas import tpu as pltpu


def _moe_body(x_ref, wr_ref, wg_ref, wu_ref, wd_ref, out_ref, comb_ref):
    e = pl.program_id(0)
    ib = pl.program_id(1)
    T, E = comb_ref.shape

    @pl.when((e == 0) & (ib == 0))
    def _init():
        x = x_ref[...]
        logits = jnp.dot(x, wr_ref[...], preferred_element_type=jnp.float32)
        aff = jax.nn.softmax(logits, axis=-1)
        idx = jax.lax.broadcasted_iota(jnp.int32, (T, E), 1)
        m1 = jnp.max(aff, axis=-1, keepdims=True)
        i1 = jnp.min(jnp.where(aff == m1, idx, E), axis=-1, keepdims=True)
        mask1 = idx == i1
        aff2 = jnp.where(mask1, -1.0, aff)
        m2 = jnp.max(aff2, axis=-1, keepdims=True)
        i2 = jnp.min(jnp.where(aff2 == m2, idx, E), axis=-1, keepdims=True)
        mask2 = idx == i2
        denom = m1 + m2
        comb_ref[...] = (
            jnp.where(mask1, m1 / denom, 0.0) + jnp.where(mask2, m2 / denom, 0.0)
        )
        out_ref[...] = jnp.zeros_like(out_ref)

    x = x_ref[...]
    gate = jnp.dot(x, wg_ref[0], preferred_element_type=jnp.float32)
    up = jnp.dot(x, wu_ref[0], preferred_element_type=jnp.float32)
    hmid = gate * jax.nn.sigmoid(gate) * up

    comb = comb_ref[...]
    idx = jax.lax.broadcasted_iota(jnp.int32, (T, E), 1)
    w_e = jnp.sum(jnp.where(idx == e, comb, 0.0), axis=-1, keepdims=True)
    hmid = hmid * w_e
    out_ref[...] += jnp.dot(hmid, wd_ref[0], preferred_element_type=jnp.float32)


def kernel(x, W_router, W_gate, W_up, W_down):
    B, S, H = x.shape
    E, _, I = W_gate.shape
    T = B * S
    IB = 512
    n_ib = I // IB

    out = pl.pallas_call(
        _moe_body,
        grid=(E, n_ib),
        in_specs=[
            pl.BlockSpec((T, None, H), lambda e, i: (0, 0, 0)),
            pl.BlockSpec((H, E), lambda e, i: (0, 0)),
            pl.BlockSpec((1, H, IB), lambda e, i: (e, 0, i)),
            pl.BlockSpec((1, H, IB), lambda e, i: (e, 0, i)),
            pl.BlockSpec((1, IB, H), lambda e, i: (e, i, 0)),
        ],
        out_specs=pl.BlockSpec((T, None, H), lambda e, i: (0, 0, 0)),
        out_shape=jax.ShapeDtypeStruct((B, S, H), x.dtype),
        scratch_shapes=[pltpu.VMEM((T, E), jnp.float32)],
        compiler_params=pltpu.CompilerParams(
            dimension_semantics=("arbitrary", "arbitrary"),
        ),
    )(x, W_router, W_gate, W_up, W_down)
    return out
